# baseline (speedup 1.0000x reference)
import functools

import jax
import jax.numpy as jnp
from jax import lax
from jax.experimental import pallas as pl
from jax.experimental.pallas import tpu as pltpu
from jax.experimental.pallas import tpu_sc as plsc

_NC = 2
_NS = 16
_NW = _NC * _NS
_L = 16


def _proj_body(x_ref, wt_ref, b_ref, h32_ref):
    h = (jnp.dot(x_ref[...], wt_ref[...], preferred_element_type=jnp.float32)
         + b_ref[...])
    blk, d = h.shape
    hr = h.reshape(blk, d // 32, 2, 16)
    u = jax.lax.bitcast_convert_type(hr, jnp.uint32)
    r = (u + jnp.uint32(0x7FFF) + ((u >> 16) & jnp.uint32(1))) >> 16
    word = r[:, :, 0, :] | (r[:, :, 1, :] << 16)
    h32_ref[...] = jax.lax.bitcast_convert_type(word, jnp.int32).reshape(
        blk, d // 2)


def _project_packed(x, wt, b2):
    n, d = x.shape
    blk = 2000
    return pl.pallas_call(
        _proj_body,
        grid=(n // blk,),
        in_specs=[
            pl.BlockSpec((blk, d), lambda i: (i, 0)),
            pl.BlockSpec((d, d), lambda i: (0, 0)),
            pl.BlockSpec((1, d), lambda i: (0, 0)),
        ],
        out_specs=pl.BlockSpec((blk, d // 2), lambda i: (i, 0)),
        out_shape=jax.ShapeDtypeStruct((n, d // 2), jnp.int32),
    )(x, wt, b2)


@functools.lru_cache(maxsize=None)
def _make_sc(n_nodes, e_total, d, c):
    epw = e_total // _NW
    g_full = c // _L
    tail = c % _L
    nchunks = epw // c
    assert nchunks % 2 == 0 and c % 8 == 0
    mesh = plsc.VectorSubcoreMesh(core_axis_name="c", subcore_axis_name="s")

    @functools.partial(
        pl.kernel,
        mesh=mesh,
        compiler_params=pltpu.CompilerParams(
            needs_layout_passes=False, use_tc_tiling_on_sc=False),
        out_type=[
            jax.ShapeDtypeStruct((e_total,), jnp.float32),
            jax.ShapeDtypeStruct((e_total, d), jnp.float32),
        ],
        scratch_types=[
            [pltpu.VMEM((c,), jnp.int32) for _ in range(2)],
            [pltpu.VMEM((c,), jnp.int32) for _ in range(2)],
            [pltpu.VMEM((c + _L, d // 2), jnp.int32) for _ in range(2)],
            [pltpu.VMEM((c + _L, d // 2), jnp.int32) for _ in range(2)],
            [pltpu.VMEM((c + _L, d), jnp.float32) for _ in range(2)],
            pltpu.VMEM((epw + _L,), jnp.float32),
            pltpu.VMEM((_L * (_L + 1),), jnp.float32),
            [pltpu.SemaphoreType.DMA for _ in range(2)],
            [pltpu.SemaphoreType.DMA for _ in range(2)],
        ],
    )
    def sc_kern(hbf_hbm, src_hbm, dst_hbm, score_out, hdst_out,
                sidx, didx, srows, drows, wrows, scv, part, gsem, wsem):
        wid = lax.axis_index("s") * _NC + lax.axis_index("c")
        base = wid * epw
        lane = lax.iota(jnp.int32, 16)
        himask = jnp.full((16,), -65536, jnp.int32)

        def issue_gathers(ci, p):
            cbase = base + ci * c
            pltpu.sync_copy(src_hbm.at[pl.ds(cbase, c)], sidx[p])
            pltpu.sync_copy(dst_hbm.at[pl.ds(cbase, c)], didx[p])
            pltpu.async_copy(hbf_hbm.at[sidx[p]], srows[p].at[pl.ds(0, c)],
                             gsem[p])
            pltpu.async_copy(hbf_hbm.at[didx[p]], drows[p].at[pl.ds(0, c)],
                             gsem[p])

        def wait_gathers(p):
            pltpu.make_async_copy(hbf_hbm.at[sidx[p]],
                                  srows[p].at[pl.ds(0, c)], gsem[p]).wait()
            pltpu.make_async_copy(hbf_hbm.at[didx[p]],
                                  drows[p].at[pl.ds(0, c)], gsem[p]).wait()

        def wait_write(p):
            pltpu.make_async_copy(wrows[p].at[pl.ds(0, c)],
                                  hdst_out.at[pl.ds(base, c)], wsem[p]).wait()

        def compute_chunk(ci, p):
            sr, dr, wr = srows[p], drows[p], wrows[p]

            def group(g):
                e0 = g * _L
                for e in range(_L):
                    acc = jnp.zeros((16,), jnp.float32)
                    for j in range(d // 32):
                        ws = sr[e0 + e, pl.ds(j * 16, 16)]
                        wd = dr[e0 + e, pl.ds(j * 16, 16)]
                        slo = plsc.bitcast(ws << 16, jnp.float32)
                        shi = plsc.bitcast(ws & himask, jnp.float32)
                        dlo = plsc.bitcast(wd << 16, jnp.float32)
                        dhi = plsc.bitcast(wd & himask, jnp.float32)
                        wr[e0 + e, pl.ds(j * 32, 16)] = dlo
                        wr[e0 + e, pl.ds(j * 32 + 16, 16)] = dhi
                        acc = acc + slo * dlo + shi * dhi
                    part[pl.ds(e * (_L + 1), 16)] = acc
                tot = jnp.zeros((16,), jnp.float32)
                lane17 = lane * (_L + 1)
                for k in range(16):
                    tot = tot + plsc.load_gather(part, [lane17 + k])
                scv[pl.ds(ci * c + e0, 16)] = 1.0 / (1.0 + jnp.exp(-tot))

            def group_body(g, carry2):
                group(g)
                return carry2

            lax.fori_loop(0, g_full, group_body, 0)
            if tail:
                group(g_full)

        for p in range(2):
            issue_gathers(p, p)

        def outer(t, carry):
            for p in range(2):
                ci = 2 * t + p
                wait_gathers(p)

                @pl.when(ci >= 2)
                def _drain_write():
                    wait_write(p)

                compute_chunk(ci, p)
                pltpu.async_copy(wrows[p].at[pl.ds(0, c)],
                                 hdst_out.at[pl.ds(base + ci * c, c)], wsem[p])

                @pl.when(ci + 2 < nchunks)
                def _prefetch():
                    issue_gathers(ci + 2, p)
            return carry

        lax.fori_loop(0, nchunks // 2, outer, 0)
        for p in range(2):
            wait_write(p)
        pltpu.sync_copy(scv.at[pl.ds(0, epw)], score_out.at[pl.ds(base, epw)])

    return sc_kern


def kernel(x, edge_index, W, b):
    e_total = edge_index.shape[1]
    d = x.shape[1]
    src = edge_index[0]
    dst = edge_index[1]
    h32 = _project_packed(x, W.T, b.reshape(1, d))
    score, h_dst = _make_sc(x.shape[0], e_total, d, 200)(h32, src, dst)
    return score.reshape(e_total, 1), h_dst, src, dst


# speedup vs baseline: 1.1801x; 1.1801x over previous
import functools

import jax
import jax.numpy as jnp
from jax import lax
from jax.experimental import pallas as pl
from jax.experimental.pallas import tpu as pltpu
from jax.experimental.pallas import tpu_sc as plsc

_NC = 2
_NS = 16
_NW = _NC * _NS
_L = 16


def _proj_body(x_ref, wt_ref, b_ref, h32_ref):
    h = (jnp.dot(x_ref[...], wt_ref[...], preferred_element_type=jnp.float32)
         + b_ref[...])
    blk, d = h.shape
    u = jax.lax.bitcast_convert_type(h, jnp.uint32)
    r = (u + jnp.uint32(0x7FFF) + ((u >> 16) & jnp.uint32(1))) >> 16
    word = r[:, : d // 2] | (r[:, d // 2:] << 16)
    h32_ref[...] = jax.lax.bitcast_convert_type(word, jnp.int32)


def _project_packed(x, wt, b2):
    n, d = x.shape
    blk = 2000
    return pl.pallas_call(
        _proj_body,
        grid=(n // blk,),
        in_specs=[
            pl.BlockSpec((blk, d), lambda i: (i, 0)),
            pl.BlockSpec((d, d), lambda i: (0, 0)),
            pl.BlockSpec((1, d), lambda i: (0, 0)),
        ],
        out_specs=pl.BlockSpec((blk, d // 2), lambda i: (i, 0)),
        out_shape=jax.ShapeDtypeStruct((n, d // 2), jnp.int32),
    )(x, wt, b2)


@functools.lru_cache(maxsize=None)
def _make_sc(n_nodes, e_total, d, c):
    epw = e_total // _NW
    g_full = c // _L
    tail = c % _L
    nchunks = epw // c
    assert nchunks % 2 == 0 and c % 8 == 0
    mesh = plsc.VectorSubcoreMesh(core_axis_name="c", subcore_axis_name="s")

    @functools.partial(
        pl.kernel,
        mesh=mesh,
        compiler_params=pltpu.CompilerParams(
            needs_layout_passes=False, use_tc_tiling_on_sc=False),
        out_type=[
            jax.ShapeDtypeStruct((e_total,), jnp.float32),
            jax.ShapeDtypeStruct((e_total, d), jnp.float32),
        ],
        scratch_types=[
            [pltpu.VMEM((c,), jnp.int32) for _ in range(2)],
            [pltpu.VMEM((c,), jnp.int32) for _ in range(2)],
            [pltpu.VMEM((c + _L, d // 2), jnp.int32) for _ in range(2)],
            [pltpu.VMEM((c + _L, d // 2), jnp.int32) for _ in range(2)],
            [pltpu.VMEM((c + _L, d), jnp.float32) for _ in range(2)],
            pltpu.VMEM((epw + _L,), jnp.float32),
            pltpu.VMEM((_L * (_L + 1),), jnp.float32),
            [pltpu.SemaphoreType.DMA for _ in range(2)],
            [pltpu.SemaphoreType.DMA for _ in range(2)],
        ],
    )
    def sc_kern(hbf_hbm, src_hbm, dst_hbm, score_out, hdst_out,
                sidx, didx, srows, drows, wrows, scv, part, gsem, wsem):
        wid = lax.axis_index("s") * _NC + lax.axis_index("c")
        base = wid * epw
        lane = lax.iota(jnp.int32, 16)
        himask = jnp.full((16,), -65536, jnp.int32)

        def issue_gathers(ci, p):
            cbase = base + ci * c
            pltpu.sync_copy(src_hbm.at[pl.ds(cbase, c)], sidx[p])
            pltpu.sync_copy(dst_hbm.at[pl.ds(cbase, c)], didx[p])
            pltpu.async_copy(hbf_hbm.at[sidx[p]], srows[p].at[pl.ds(0, c)],
                             gsem[p])
            pltpu.async_copy(hbf_hbm.at[didx[p]], drows[p].at[pl.ds(0, c)],
                             gsem[p])

        def wait_gathers(p):
            pltpu.make_async_copy(hbf_hbm.at[sidx[p]],
                                  srows[p].at[pl.ds(0, c)], gsem[p]).wait()
            pltpu.make_async_copy(hbf_hbm.at[didx[p]],
                                  drows[p].at[pl.ds(0, c)], gsem[p]).wait()

        def wait_write(p):
            pltpu.make_async_copy(wrows[p].at[pl.ds(0, c)],
                                  hdst_out.at[pl.ds(base, c)], wsem[p]).wait()

        def compute_chunk(ci, p):
            sr, dr, wr = srows[p], drows[p], wrows[p]

            def group(g):
                e0 = g * _L
                for e in range(_L):
                    acc = jnp.zeros((16,), jnp.float32)
                    for j in range(d // 32):
                        ws = sr[e0 + e, pl.ds(j * 16, 16)]
                        wd = dr[e0 + e, pl.ds(j * 16, 16)]
                        slo = plsc.bitcast(ws << 16, jnp.float32)
                        shi = plsc.bitcast(ws & himask, jnp.float32)
                        dlo = plsc.bitcast(wd << 16, jnp.float32)
                        dhi = plsc.bitcast(wd & himask, jnp.float32)
                        wr[e0 + e, pl.ds(j * 32, 16)] = dlo
                        wr[e0 + e, pl.ds(j * 32 + 16, 16)] = dhi
                        acc = acc + slo * dlo + shi * dhi
                    part[pl.ds(e * (_L + 1), 16)] = acc
                tot = jnp.zeros((16,), jnp.float32)
                lane17 = lane * (_L + 1)
                for k in range(16):
                    tot = tot + plsc.load_gather(part, [lane17 + k])
                scv[pl.ds(ci * c + e0, 16)] = 1.0 / (1.0 + jnp.exp(-tot))

            def group_body(g, carry2):
                group(g)
                return carry2

            lax.fori_loop(0, g_full, group_body, 0)
            if tail:
                group(g_full)

        for p in range(2):
            issue_gathers(p, p)

        def outer(t, carry):
            for p in range(2):
                ci = 2 * t + p
                wait_gathers(p)

                @pl.when(ci >= 2)
                def _drain_write():
                    wait_write(p)

                compute_chunk(ci, p)
                pltpu.async_copy(wrows[p].at[pl.ds(0, c)],
                                 hdst_out.at[pl.ds(base + ci * c, c)], wsem[p])

                @pl.when(ci + 2 < nchunks)
                def _prefetch():
                    issue_gathers(ci + 2, p)
            return carry

        lax.fori_loop(0, nchunks // 2, outer, 0)
        for p in range(2):
            wait_write(p)
        pltpu.sync_copy(scv.at[pl.ds(0, epw)], score_out.at[pl.ds(base, epw)])

    return sc_kern


def kernel(x, edge_index, W, b):
    e_total = edge_index.shape[1]
    d = x.shape[1]
    src = edge_index[0]
    dst = edge_index[1]
    lo = [32 * (k // 16) + k % 16 for k in range(d // 2)]
    perm = jnp.array(lo + [p + 16 for p in lo], jnp.int32)
    h32 = _project_packed(x, W.T[:, perm], b[perm].reshape(1, d))
    score, h_dst = _make_sc(x.shape[0], e_total, d, 200)(h32, src, dst)
    return score.reshape(e_total, 1), h_dst, src, dst


# speedup vs baseline: 1.3893x; 1.1773x over previous
import functools

import jax
import jax.numpy as jnp
from jax import lax
from jax.experimental import pallas as pl
from jax.experimental.pallas import tpu as pltpu
from jax.experimental.pallas import tpu_sc as plsc

_NC = 2
_NS = 16
_NW = _NC * _NS
_L = 16


def _proj_body(x_ref, wt_ref, b_ref, h_ref):
    h_ref[...] = (
        jnp.dot(x_ref[...], wt_ref[...], preferred_element_type=jnp.float32)
        + b_ref[...]
    )


def _project(x, wt, b2):
    n, d = x.shape
    blk = 2000
    return pl.pallas_call(
        _proj_body,
        grid=(n // blk,),
        in_specs=[
            pl.BlockSpec((blk, d), lambda i: (i, 0)),
            pl.BlockSpec((d, d), lambda i: (0, 0)),
            pl.BlockSpec((1, d), lambda i: (0, 0)),
        ],
        out_specs=pl.BlockSpec((blk, d), lambda i: (i, 0)),
        out_shape=jax.ShapeDtypeStruct((n, d), jnp.float32),
    )(x, wt, b2)


@functools.lru_cache(maxsize=None)
def _make_sc(n_nodes, e_total, d, c):
    epw = e_total // _NW
    g_full = c // _L
    tail = c % _L
    nchunks = epw // c
    assert nchunks % 2 == 0 and c % 8 == 0
    mesh = plsc.VectorSubcoreMesh(core_axis_name="c", subcore_axis_name="s")

    @functools.partial(
        pl.kernel,
        mesh=mesh,
        compiler_params=pltpu.CompilerParams(
            needs_layout_passes=False, use_tc_tiling_on_sc=False),
        out_type=[
            jax.ShapeDtypeStruct((e_total,), jnp.float32),
            jax.ShapeDtypeStruct((e_total, d), jnp.float32),
        ],
        scratch_types=[
            [pltpu.VMEM((c,), jnp.int32) for _ in range(2)],
            [pltpu.VMEM((c,), jnp.int32) for _ in range(2)],
            [pltpu.VMEM((c + _L, d), jnp.float32) for _ in range(2)],
            [pltpu.VMEM((c + _L, d), jnp.float32) for _ in range(2)],
            pltpu.VMEM((epw + _L,), jnp.float32),
            pltpu.VMEM((_L * (_L + 1),), jnp.float32),
            [pltpu.SemaphoreType.DMA for _ in range(2)],
            [pltpu.SemaphoreType.DMA for _ in range(2)],
        ],
    )
    def sc_kern(h_hbm, src_hbm, dst_hbm, score_out, hdst_out,
                sidx, didx, srows, drows, scv, part, gsem, wsem):
        wid = lax.axis_index("s") * _NC + lax.axis_index("c")
        base = wid * epw
        lane = lax.iota(jnp.int32, 16)

        def issue_gathers(ci, p):
            cbase = base + ci * c
            pltpu.sync_copy(src_hbm.at[pl.ds(cbase, c)], sidx[p])
            pltpu.sync_copy(dst_hbm.at[pl.ds(cbase, c)], didx[p])
            pltpu.async_copy(h_hbm.at[sidx[p]], srows[p].at[pl.ds(0, c)],
                             gsem[p])
            pltpu.async_copy(h_hbm.at[didx[p]], drows[p].at[pl.ds(0, c)],
                             gsem[p])

        def wait_gathers(p):
            pltpu.make_async_copy(h_hbm.at[sidx[p]],
                                  srows[p].at[pl.ds(0, c)], gsem[p]).wait()
            pltpu.make_async_copy(h_hbm.at[didx[p]],
                                  drows[p].at[pl.ds(0, c)], gsem[p]).wait()

        def compute_chunk(ci, p):
            sr, dr = srows[p], drows[p]

            def group(g):
                e0 = g * _L
                for e in range(_L):
                    acc = (sr[e0 + e, pl.ds(0, 16)]
                           * dr[e0 + e, pl.ds(0, 16)])
                    for j in range(1, d // 16):
                        acc = acc + (sr[e0 + e, pl.ds(j * 16, 16)]
                                     * dr[e0 + e, pl.ds(j * 16, 16)])
                    part[pl.ds(e * (_L + 1), 16)] = acc
                tot = jnp.zeros((16,), jnp.float32)
                lane17 = lane * (_L + 1)
                for k in range(16):
                    tot = tot + plsc.load_gather(part, [lane17 + k])
                scv[pl.ds(ci * c + e0, 16)] = 1.0 / (1.0 + jnp.exp(-tot))

            def group_body(g, carry2):
                group(g)
                return carry2

            lax.fori_loop(0, g_full, group_body, 0)
            if tail:
                group(g_full)

        for p in range(2):
            issue_gathers(p, p)

        def outer(t, carry):
            for p in range(2):
                ci = 2 * t + p
                wait_gathers(p)
                pltpu.async_copy(drows[p].at[pl.ds(0, c)],
                                 hdst_out.at[pl.ds(base + ci * c, c)], wsem[p])
                compute_chunk(ci, p)

                @pl.when(ci + 2 < nchunks)
                def _prefetch():
                    pltpu.make_async_copy(
                        drows[p].at[pl.ds(0, c)],
                        hdst_out.at[pl.ds(base, c)], wsem[p]).wait()
                    issue_gathers(ci + 2, p)
            return carry

        lax.fori_loop(0, nchunks // 2, outer, 0)
        for p in range(2):
            pltpu.make_async_copy(
                drows[p].at[pl.ds(0, c)],
                hdst_out.at[pl.ds(base, c)], wsem[p]).wait()
        pltpu.sync_copy(scv.at[pl.ds(0, epw)], score_out.at[pl.ds(base, epw)])

    return sc_kern


def kernel(x, edge_index, W, b):
    e_total = edge_index.shape[1]
    d = x.shape[1]
    src = edge_index[0]
    dst = edge_index[1]
    h = _project(x, W.T, b.reshape(1, d))
    score, h_dst = _make_sc(x.shape[0], e_total, d, 200)(h, src, dst)
    return score.reshape(e_total, 1), h_dst, src, dst
